# Initial kernel scaffold; baseline (speedup 1.0000x reference)
import jax
import jax.numpy as jnp
from jax.experimental import pallas as pl


def kernel(x, A_indices, A_values, W, b):
    raise NotImplementedError("write your pallas kernel here")


# speedup vs baseline: 5.3827x; 5.3827x over previous
import functools

import jax
import jax.numpy as jnp
from jax import lax
from jax.experimental import pallas as pl
from jax.experimental.pallas import tpu as pltpu
from jax.experimental.pallas import tpu_sc as plsc

N = 10000
E = 320000
D = 128

NC = 2
NS = 16
L = 16

C = 128
NCHUNK = E // C
NW = NC * NS
CH_BASE = NCHUNK // NW
CH_REM = NCHUNK % NW
ROWS_A = (N // NS) // 8 * 8
ROWS_LAST = N - (NS - 1) * ROWS_A


def _matmul_body(x_ref, wt_ref, b_ref, h_ref):
    h_ref[...] = (
        jnp.dot(x_ref[...], wt_ref[...], preferred_element_type=jnp.float32)
        + b_ref[...]
    )


def _dense_h(x, wt, b2d):
    grid = 10
    blk = N // grid
    return pl.pallas_call(
        _matmul_body,
        grid=(grid,),
        in_specs=[
            pl.BlockSpec((blk, D), lambda i: (i, 0)),
            pl.BlockSpec((D, D), lambda i: (0, 0)),
            pl.BlockSpec((1, D), lambda i: (0, 0)),
        ],
        out_specs=pl.BlockSpec((blk, D), lambda i: (i, 0)),
        out_shape=jax.ShapeDtypeStruct((N, D), jnp.float32),
    )(x, wt, b2d)


def _add_body(a_ref, b_ref, o_ref):
    o_ref[...] = a_ref[...] + b_ref[...]


def _combine(partials):
    grid = 10
    blk = N // grid
    return pl.pallas_call(
        _add_body,
        grid=(grid,),
        in_specs=[
            pl.BlockSpec((blk, D), lambda i: (i, 0)),
            pl.BlockSpec((blk, D), lambda i: (i + grid, 0)),
        ],
        out_specs=pl.BlockSpec((blk, D), lambda i: (i, 0)),
        out_shape=jax.ShapeDtypeStruct((N, D), jnp.float32),
    )(partials, partials)


def _sc_body(h_hbm, rows_hbm, cols_hbm, vals_hbm, out_hbm,
             cols_idx, rows_idx, vals_v, rows_buf, acc_sh, sem):
    c = lax.axis_index("c")
    s = lax.axis_index("s")
    wid = s * NC + c

    zeros16 = jnp.zeros((L,), jnp.float32)

    def _zero_row(r, _):
        for q in range(D // L):
            rows_buf[r, pl.ds(q * L, L)] = zeros16
        return 0

    lax.fori_loop(0, C, _zero_row, 0)

    acc_base = s * ROWS_A

    for k in range(ROWS_A // C):
        pltpu.sync_copy(rows_buf, acc_sh.at[pl.ds(acc_base + k * C, C)])
    tail0 = ROWS_A - (ROWS_A // C) * C
    pltpu.sync_copy(rows_buf.at[pl.ds(0, tail0)],
                    acc_sh.at[pl.ds(acc_base + (ROWS_A // C) * C, tail0)])

    @pl.when(s == NS - 1)
    def _zero_extra():
        extra = ROWS_LAST - ROWS_A
        pltpu.sync_copy(rows_buf.at[pl.ds(0, extra)],
                        acc_sh.at[pl.ds(acc_base + ROWS_A, extra)])

    plsc.subcore_barrier()

    start = wid * CH_BASE + jnp.minimum(wid, CH_REM)
    count = CH_BASE + jnp.where(wid < CH_REM, 1, 0)

    def _chunk(j, _):
        base = (start + j) * C
        pltpu.sync_copy(cols_hbm.at[pl.ds(base, C)], cols_idx)
        pltpu.sync_copy(rows_hbm.at[pl.ds(base, C)], rows_idx)
        pltpu.sync_copy(vals_hbm.at[pl.ds(base, C)], vals_v)
        pltpu.async_copy(h_hbm.at[cols_idx], rows_buf, sem).wait()

        def _group(g, _):
            v16 = vals_v[pl.ds(g * L, L)]
            for e in range(L):
                r = g * L + e
                bval = jnp.broadcast_to(v16[e], (L,))
                for q in range(D // L):
                    sl = pl.ds(q * L, L)
                    rows_buf[r, sl] = rows_buf[r, sl] * bval
            return 0

        lax.fori_loop(0, C // L, _group, 0)

        pltpu.sync_copy(rows_buf, acc_sh.at[rows_idx], add=True)
        return 0

    lax.fori_loop(0, count, _chunk, 0)
    plsc.subcore_barrier()

    out_base = c * N + acc_base
    for k in range(ROWS_A // C):
        pltpu.sync_copy(acc_sh.at[pl.ds(acc_base + k * C, C)],
                        out_hbm.at[pl.ds(out_base + k * C, C)])
    pltpu.sync_copy(acc_sh.at[pl.ds(acc_base + (ROWS_A // C) * C, tail0)],
                    out_hbm.at[pl.ds(out_base + (ROWS_A // C) * C, tail0)])

    @pl.when(s == NS - 1)
    def _write_extra():
        extra = ROWS_LAST - ROWS_A
        pltpu.sync_copy(acc_sh.at[pl.ds(acc_base + ROWS_A, extra)],
                        out_hbm.at[pl.ds(out_base + ROWS_A, extra)])


@functools.partial(jax.jit, static_argnums=())
def _sc_aggregate(h, rows, cols, vals):
    mesh = plsc.VectorSubcoreMesh(core_axis_name="c", subcore_axis_name="s")
    return pl.kernel(
        _sc_body,
        out_type=jax.ShapeDtypeStruct((NC * N, D), jnp.float32),
        mesh=mesh,
        scratch_types=[
            pltpu.VMEM((C,), jnp.int32),
            pltpu.VMEM((C,), jnp.int32),
            pltpu.VMEM((C,), jnp.float32),
            pltpu.VMEM((C, D), jnp.float32),
            pltpu.VMEM_SHARED((N, D), jnp.float32),
            pltpu.SemaphoreType.DMA,
        ],
    )(h, rows, cols, vals)


def kernel(x, A_indices, A_values, W, b):
    rows = A_indices[0]
    cols = A_indices[1]
    h = _dense_h(x, W.T, b.reshape(1, D))
    partials = _sc_aggregate(h, rows, cols, A_values)
    return _combine(partials)
